# Initial kernel scaffold; baseline (speedup 1.0000x reference)
import jax
import jax.numpy as jnp
from jax.experimental import pallas as pl


def kernel(data, segment_ids):
    raise NotImplementedError("write your pallas kernel here")


# speedup vs baseline: 3.1216x; 3.1216x over previous
import functools

import jax
import jax.numpy as jnp
from jax import lax
from jax.experimental import pallas as pl
from jax.experimental.pallas import tpu as pltpu
from jax.experimental.pallas import tpu_sc as plsc

N = 320000
D = 128
S = 10000
NC = 2
NT = 16
H = S // NC
B = 128
RPT = N // NT
NB = RPT // B
TAIL_BASE = NT * NB * B
TAIL_BATCHES = (N - TAIL_BASE) // B
ACC_ROWS = 5120
ZPT = ACC_ROWS // NT
WCHUNKS = -(-H // B)


def _seg_sum_body(data_hbm, ids_hbm, out_hbm, ids_buf, ids_x, data_buf,
                  idx_buf, zbuf, acc):
    cid = lax.axis_index("c")
    tid = lax.axis_index("s")
    lo = cid * H
    hi = lo + H

    def zrow(i, _):
        def zcol(j, _):
            zbuf[i, pl.ds(j * 16, 16)] = jnp.zeros((16,), jnp.float32)
            return 0
        return lax.fori_loop(0, D // 16, zcol, 0)
    lax.fori_loop(0, B, zrow, 0)
    z0 = tid * ZPT
    pltpu.sync_copy(zbuf, acc.at[pl.ds(z0, B)])
    pltpu.sync_copy(zbuf, acc.at[pl.ds(z0 + B, B)])
    pltpu.sync_copy(zbuf, acc.at[pl.ds(z0 + (ZPT - B), B)])

    pltpu.sync_copy(ids_hbm.at[pl.ds(tid * NB * B, NB * B)], ids_buf)

    plsc.subcore_barrier()

    def accumulate(ids_ref, o, row0):
        first = ids_ref[pl.ds(o, 16)][0]
        last = ids_ref[pl.ds(o + B - 16, 16)][15]

        @pl.when((first < hi) & (last >= lo))
        def _():
            pltpu.sync_copy(data_hbm.at[pl.ds(row0, B)], data_buf)

            def cchunk(j, _):
                ids16 = ids_ref[pl.ds(o + j * 16, 16)]
                rel = ids16 - lo
                ok = (rel >= 0) & (rel < H)
                idx_buf[pl.ds(j * 16, 16)] = jnp.where(ok, rel, H)
                return 0
            lax.fori_loop(0, B // 16, cchunk, 0)
            pltpu.sync_copy(data_buf, acc.at[idx_buf], add=True)

    def batch(k, _):
        accumulate(ids_buf, k * B, tid * NB * B + k * B)
        return 0
    lax.fori_loop(0, NB, batch, 0)

    @pl.when(tid < TAIL_BATCHES)
    def _():
        row0 = TAIL_BASE + tid * B
        pltpu.sync_copy(ids_hbm.at[pl.ds(row0, B)], ids_x)
        accumulate(ids_x, 0, row0)

    plsc.subcore_barrier()

    def wout(c, _):
        chunk = tid + NT * c

        @pl.when(chunk < WCHUNKS)
        def _():
            st = jnp.minimum(chunk * B, H - B)
            pltpu.sync_copy(acc.at[pl.ds(st, B)],
                            out_hbm.at[pl.ds(lo + st, B)])
        return 0
    lax.fori_loop(0, -(-WCHUNKS // NT), wout, 0)


_seg_sum = pl.kernel(
    _seg_sum_body,
    out_type=jax.ShapeDtypeStruct((S, D), jnp.float32),
    mesh=plsc.VectorSubcoreMesh(core_axis_name="c", subcore_axis_name="s"),
    scratch_types=[
        pltpu.VMEM((NB * B,), jnp.int32),
        pltpu.VMEM((B,), jnp.int32),
        pltpu.VMEM((B, D), jnp.float32),
        pltpu.VMEM((B,), jnp.int32),
        pltpu.VMEM((B, D), jnp.float32),
        pltpu.VMEM_SHARED((ACC_ROWS, D), jnp.float32),
    ],
)


def kernel(data, segment_ids):
    return _seg_sum(data, segment_ids)


# speedup vs baseline: 6.7669x; 2.1678x over previous
import jax
import jax.numpy as jnp
from jax import lax
from jax.experimental import pallas as pl
from jax.experimental.pallas import tpu as pltpu
from jax.experimental.pallas import tpu_sc as plsc

N = 320000
D = 128
S = 10000
NC = 2
NT = 16
BS = 40
H = S // NC
B = 128
RPT = N // NT
NB = RPT // B
TAIL_BASE = NT * NB * B
TAIL_BATCHES = (N - TAIL_BASE) // B
ACC_ROWS = 5120
ZPT = ACC_ROWS // NT
WCHUNKS = H // BS
DIV_M = (1 << 21) // BS + 1


def _div_bs(x):
    return (x * DIV_M) >> 21


def _seg_sum_body(data_hbm, ids_hbm, out_hbm, ids_buf, ids_x, data0, data1,
                  idx0, idx1, zbuf, acc, sem_in, sem_s0, sem_s1):
    cid = lax.axis_index("c")
    tid = lax.axis_index("s")

    def zrow(i, _):
        def zcol(j, _):
            zbuf[i, pl.ds(j * 16, 16)] = jnp.zeros((16,), jnp.float32)
            return 0
        return lax.fori_loop(0, D // 16, zcol, 0)
    lax.fori_loop(0, B, zrow, 0)
    z0 = tid * ZPT
    pltpu.sync_copy(zbuf, acc.at[pl.ds(z0, B)])
    pltpu.sync_copy(zbuf, acc.at[pl.ds(z0 + B, B)])
    pltpu.sync_copy(zbuf, acc.at[pl.ds(z0 + (ZPT - B), B)])

    pltpu.sync_copy(ids_hbm.at[pl.ds(tid * NB * B, NB * B)], ids_buf)

    plsc.subcore_barrier()

    def gen_idx(ids_ref, o, idx_b):
        def cchunk(j, _):
            ids16 = ids_ref[pl.ds(o + j * 16, 16)]
            blk = _div_bs(ids16)
            rel = (blk >> 1) * BS + (ids16 - blk * BS)
            own = (blk & 1) == cid
            idx_b[pl.ds(j * 16, 16)] = jnp.where(own, rel, H)
            return 0
        lax.fori_loop(0, B // 16, cchunk, 0)

    def owned_cond(ids_ref, o):
        first = ids_ref[pl.ds(o, 16)][0]
        last = ids_ref[pl.ds(o + B - 16, 16)][15]
        fb = _div_bs(first)
        lb = _div_bs(last)
        return (lb > fb) | ((fb & 1) == cid)

    def process(data_b, idx_b, sem_s, pend, o, row0):
        @pl.when(pend == 1)
        def _():
            pltpu.make_async_copy(data_b, acc.at[idx_b], sem_s).wait()
        desc = pltpu.async_copy(data_hbm.at[pl.ds(row0, B)], data_b, sem_in)
        gen_idx(ids_buf, o, idx_b)
        desc.wait()
        pltpu.async_copy(data_b, acc.at[idx_b], sem_s, add=True)

    def batch(k, carry):
        p, pend0, pend1 = carry
        o = k * B
        row0 = tid * NB * B + o
        cond = owned_cond(ids_buf, o)

        @pl.when(cond & (p == 0))
        def _():
            process(data0, idx0, sem_s0, pend0, o, row0)

        @pl.when(cond & (p == 1))
        def _():
            process(data1, idx1, sem_s1, pend1, o, row0)

        p2 = jnp.where(cond, 1 - p, p)
        pend0_2 = jnp.where(cond & (p == 0), 1, pend0)
        pend1_2 = jnp.where(cond & (p == 1), 1, pend1)
        return p2, pend0_2, pend1_2

    p, pend0, pend1 = lax.fori_loop(
        0, NB, batch, (jnp.int32(0), jnp.int32(0), jnp.int32(0)))

    @pl.when(pend0 == 1)
    def _():
        pltpu.make_async_copy(data0, acc.at[idx0], sem_s0).wait()

    @pl.when(pend1 == 1)
    def _():
        pltpu.make_async_copy(data1, acc.at[idx1], sem_s1).wait()

    @pl.when(tid < TAIL_BATCHES)
    def _():
        row0 = TAIL_BASE + tid * B
        pltpu.sync_copy(ids_hbm.at[pl.ds(row0, B)], ids_x)

        @pl.when(owned_cond(ids_x, 0))
        def _():
            pltpu.sync_copy(data_hbm.at[pl.ds(row0, B)], data0)
            gen_idx(ids_x, 0, idx0)
            pltpu.sync_copy(data0, acc.at[idx0], add=True)

    plsc.subcore_barrier()

    def wout(c, _):
        chunk = tid + NT * c

        @pl.when(chunk < WCHUNKS)
        def _():
            pltpu.sync_copy(acc.at[pl.ds(chunk * BS, BS)],
                            out_hbm.at[pl.ds((2 * chunk + cid) * BS, BS)])
        return 0
    lax.fori_loop(0, -(-WCHUNKS // NT), wout, 0)


_seg_sum = pl.kernel(
    _seg_sum_body,
    out_type=jax.ShapeDtypeStruct((S, D), jnp.float32),
    mesh=plsc.VectorSubcoreMesh(core_axis_name="c", subcore_axis_name="s"),
    scratch_types=[
        pltpu.VMEM((NB * B,), jnp.int32),
        pltpu.VMEM((B,), jnp.int32),
        pltpu.VMEM((B, D), jnp.float32),
        pltpu.VMEM((B, D), jnp.float32),
        pltpu.VMEM((B,), jnp.int32),
        pltpu.VMEM((B,), jnp.int32),
        pltpu.VMEM((B, D), jnp.float32),
        pltpu.VMEM_SHARED((ACC_ROWS, D), jnp.float32),
        pltpu.SemaphoreType.DMA,
        pltpu.SemaphoreType.DMA,
        pltpu.SemaphoreType.DMA,
    ],
)


def kernel(data, segment_ids):
    return _seg_sum(data, segment_ids)
